# baseline (speedup 1.0000x reference)
import functools

import jax
import jax.numpy as jnp
from jax import lax
from jax.experimental import pallas as pl
from jax.experimental.pallas import tpu as pltpu
from jax.experimental.pallas import tpu_sc as plsc

N_NODES = 10000
N_PAD = 10240
E_EDGES = 320000
CHUNKS_PER_TILE = 160
E_PAD = 16 * CHUNKS_PER_TILE * 128
ROWS_PER_TILE = N_PAD // 16
NBUF = 4


@functools.cache
def _mesh():
    return plsc.VectorSubcoreMesh(core_axis_name="c", subcore_axis_name="s")


@functools.cache
def _make_prop(width, k0):
    nb = 2 if width == 128 else 4
    stage_src = width != 128
    k1 = CHUNKS_PER_TILE - k0
    ncores = 1 if k1 == 0 else 2

    scratch = [
        pltpu.VMEM((max(k0, k1), 128) if stage_src else (4, 128), jnp.int32),
        pltpu.VMEM((max(k0, k1), 128) if stage_src else (4, 128), jnp.int32),
        pltpu.VMEM((nb, 128, width), jnp.float32),
        pltpu.VMEM_SHARED((N_PAD, width), jnp.float32),
    ] + [pltpu.SemaphoreType.DMA] * (2 * nb + (0 if stage_src else 8))

    @functools.partial(
        pl.kernel,
        out_type=jax.ShapeDtypeStruct((ncores, N_PAD, width), jnp.float32),
        mesh=_mesh(),
        compiler_params=pltpu.CompilerParams(use_tc_tiling_on_sc=False),
        scratch_types=scratch,
    )
    def prop(src_hbm, dst_hbm, h_hbm, z_hbm, out_hbm, sidx, didx, rows,
             acc_sh, *sems):
        gsem = sems[:nb]
        ssem = sems[nb:2 * nb]
        isem = sems[2 * nb:2 * nb + 4]
        dsem = sems[2 * nb + 4:]
        c = lax.axis_index("c")
        s = lax.axis_index("s")
        my_rows = s * ROWS_PER_TILE
        if ncores == 1:
            cbase = s * k0
            nch = k0
        else:
            cbase = jnp.where(c == 0, s * k0, 16 * k0 + s * k1)
            nch = jnp.where(c == 0, k0, k1)

        def stage():
            if stage_src:
                pltpu.sync_copy(dst_hbm.at[pl.ds(cbase, k1)],
                                didx.at[pl.ds(0, k1)])
                pltpu.sync_copy(src_hbm.at[pl.ds(cbase, k1)],
                                sidx.at[pl.ds(0, k1)])

                @pl.when(c == 0)
                def _():
                    pltpu.sync_copy(dst_hbm.at[pl.ds(cbase + k1, k0 - k1)],
                                    didx.at[pl.ds(k1, k0 - k1)])
                    pltpu.sync_copy(src_hbm.at[pl.ds(cbase + k1, k0 - k1)],
                                    sidx.at[pl.ds(k1, k0 - k1)])
            else:
                pltpu.sync_copy(src_hbm.at[pl.ds(cbase, 2)],
                                sidx.at[pl.ds(0, 2)])
                pltpu.sync_copy(dst_hbm.at[pl.ds(cbase, 2)],
                                didx.at[pl.ds(0, 2)])
                for u in (2, 3):
                    pltpu.async_copy(src_hbm.at[cbase + u], sidx.at[u],
                                     isem[u])
                    pltpu.async_copy(dst_hbm.at[cbase + u], didx.at[u],
                                     dsem[u])

            pltpu.sync_copy(z_hbm, rows.at[0])
            for z in range(ROWS_PER_TILE // 128):
                pltpu.sync_copy(rows.at[0],
                                acc_sh.at[pl.ds(my_rows + z * 128, 128)])
            plsc.subcore_barrier()

        def wait_rows(sem, b):
            pltpu.make_async_copy(h_hbm.at[pl.ds(0, 128)], rows.at[b],
                                  sem).wait()

        def wait_idx(u):
            pltpu.make_async_copy(src_hbm.at[cbase], sidx.at[u],
                                  isem[u]).wait()

        def wait_didx(u):
            pltpu.make_async_copy(dst_hbm.at[cbase], didx.at[u],
                                  dsem[u]).wait()

        def gather(jj, u, b):
            src_idx = sidx.at[jj] if stage_src else sidx.at[u]
            pltpu.async_copy(h_hbm.at[src_idx], rows.at[b], gsem[b])

        unroll = nb if stage_src else 4

        def body(i, _):
            for u in range(unroll):
                j = i * unroll + u
                b = u % nb
                wait_rows(gsem[b], b)
                if not stage_src:
                    @pl.when(j + 4 < nch)
                    def _():
                        pltpu.async_copy(src_hbm.at[cbase + j + 4],
                                         sidx.at[u], isem[u])
                    if u >= 2:
                        wait_didx(u)
                    else:
                        @pl.when(j >= 4)
                        def _():
                            wait_didx(u)
                dchunk = didx.at[j] if stage_src else didx.at[u]
                pltpu.async_copy(rows.at[b], acc_sh.at[dchunk],
                                 ssem[b], add=True)

                @pl.when(j + nb < nch)
                def _():
                    wait_rows(ssem[b], b)
                    if not stage_src:
                        @pl.when(j + 4 < nch)
                        def _():
                            pltpu.async_copy(dst_hbm.at[cbase + j + 4],
                                             didx.at[u], dsem[u])
                        wait_idx((u + nb) % 4)
                    gather(j + nb, (u + nb) % 4, b)

            return 0

        def run():
            stage()
            for b in range(nb):
                gather(b, b, b)
            lax.fori_loop(0, nch // unroll, body, 0)
            for b in range(nb):
                wait_rows(ssem[b], b)
            plsc.subcore_barrier()

            for z in range(ROWS_PER_TILE // 128):
                r0 = my_rows + z * 128
                pltpu.sync_copy(acc_sh.at[pl.ds(r0, 128)],
                                out_hbm.at[c, pl.ds(r0, 128)])

        if ncores == 1:
            pl.when(c == 0)(run)
        else:
            run()

    return prop


@functools.cache
def _make_deg(k0):
    k1 = CHUNKS_PER_TILE - k0

    @functools.partial(
        pl.kernel,
        out_type=jax.ShapeDtypeStruct((2, N_PAD, 16), jnp.float32),
        mesh=_mesh(),
        compiler_params=pltpu.CompilerParams(use_tc_tiling_on_sc=False),
        scratch_types=[
            pltpu.VMEM((max(k0, k1), 128), jnp.int32),
            pltpu.VMEM((128, 16), jnp.float32),
            pltpu.VMEM_SHARED((N_PAD, 16), jnp.float32),
        ] + [pltpu.SemaphoreType.DMA] * NBUF,
    )
    def deg_kernel(dst_hbm, const_hbm, out_hbm, didx, rows, acc_sh, *ssem):
        c = lax.axis_index("c")
        s = lax.axis_index("s")
        my_rows = s * ROWS_PER_TILE
        cbase = jnp.where(c == 0, s * k0, 16 * k0 + s * k1)
        nch = jnp.where(c == 0, k0, k1)

        pltpu.sync_copy(dst_hbm.at[pl.ds(cbase, k1)], didx.at[pl.ds(0, k1)])

        @pl.when(c == 0)
        def _():
            pltpu.sync_copy(dst_hbm.at[pl.ds(cbase + k1, k0 - k1)],
                            didx.at[pl.ds(k1, k0 - k1)])

        pltpu.sync_copy(const_hbm.at[0], rows)
        for z in range(ROWS_PER_TILE // 128):
            pltpu.sync_copy(rows, acc_sh.at[pl.ds(my_rows + z * 128, 128)])
        plsc.subcore_barrier()

        pltpu.sync_copy(const_hbm.at[1], rows)

        def wait(sem):
            pltpu.make_async_copy(out_hbm.at[c, pl.ds(0, 128)], rows,
                                  sem).wait()

        def body(i, _):
            for b in range(NBUF):
                j = i * NBUF + b

                @pl.when(j >= NBUF)
                def _():
                    wait(ssem[b])

                pltpu.async_copy(rows, acc_sh.at[didx.at[j]], ssem[b],
                                 add=True)
            return 0

        lax.fori_loop(0, nch // NBUF, body, 0)
        for b in range(NBUF):
            wait(ssem[b])
        plsc.subcore_barrier()

        for z in range(ROWS_PER_TILE // 128):
            r0 = my_rows + z * 128
            pltpu.sync_copy(acc_sh.at[pl.ds(r0, 128)],
                            out_hbm.at[c, pl.ds(r0, 128)])

    return deg_kernel


def _tc_a_body(x_ref, w1_ref, degp_ref, hs1_ref, dinv_ref):
    t = degp_ref[0] + degp_ref[1]
    deg = t[:, 0:1] + 1.0
    dinv = lax.rsqrt(deg)
    h = jnp.dot(x_ref[...], w1_ref[...], preferred_element_type=jnp.float32)
    hs1_ref[...] = h * dinv
    dinv_ref[...] = dinv


def _tc_b_body(accp_ref, hs1_ref, dinv_ref, b1_ref, w2_ref, hs2_ref):
    t = accp_ref[0] + hs1_ref[...]
    out1 = dinv_ref[...] * t + b1_ref[...]
    h = jnp.maximum(out1, 0.0)
    h2 = jnp.dot(h, w2_ref[...], preferred_element_type=jnp.float32)
    hs2_ref[...] = h2 * dinv_ref[...]


def _tc_c_body(accp_ref, hs2_ref, dinv_ref, b2_ref, out_ref):
    t = accp_ref[0] + accp_ref[1] + hs2_ref[...]
    out2 = dinv_ref[...] * t + b2_ref[...]
    m = jnp.max(out2, axis=1, keepdims=True)
    e = jnp.exp(out2 - m)
    lse = jnp.log(jnp.sum(e, axis=1, keepdims=True))
    out_ref[...] = out2 - m - lse


_MB = 1000


def kernel(x, edge_index, W1, b1, W2, b2):
    n = N_NODES
    pad = E_PAD - E_EDGES
    src = jnp.concatenate([edge_index[0], jnp.zeros((pad,), jnp.int32)])
    dst = jnp.concatenate([edge_index[1], jnp.full((pad,), n, jnp.int32)])
    src2 = src.reshape(E_PAD // 128, 128)
    dst2 = dst.reshape(E_PAD // 128, 128)

    zeros16 = jnp.zeros((128, 16), jnp.float32)
    deg_const = jnp.stack([zeros16, zeros16.at[:, 0].set(1.0)])
    degp = _make_deg(120)(dst2, deg_const)

    grid = (n // _MB,)
    hs1, dinv = pl.pallas_call(
        _tc_a_body,
        grid=grid,
        in_specs=[
            pl.BlockSpec((_MB, 128), lambda i: (i, 0)),
            pl.BlockSpec((128, 128), lambda i: (0, 0)),
            pl.BlockSpec((2, _MB, 16), lambda i: (0, i, 0)),
        ],
        out_specs=[
            pl.BlockSpec((_MB, 128), lambda i: (i, 0)),
            pl.BlockSpec((_MB, 1), lambda i: (i, 0)),
        ],
        out_shape=[
            jax.ShapeDtypeStruct((n, 128), jnp.float32),
            jax.ShapeDtypeStruct((n, 1), jnp.float32),
        ],
    )(x, W1, degp)

    acc1 = _make_prop(128, 160)(src2, dst2, hs1, jnp.zeros((128, 128), jnp.float32))

    hs2 = pl.pallas_call(
        _tc_b_body,
        grid=grid,
        in_specs=[
            pl.BlockSpec((1, _MB, 128), lambda i: (0, i, 0)),
            pl.BlockSpec((_MB, 128), lambda i: (i, 0)),
            pl.BlockSpec((_MB, 1), lambda i: (i, 0)),
            pl.BlockSpec((1, 128), lambda i: (0, 0)),
            pl.BlockSpec((128, 16), lambda i: (0, 0)),
        ],
        out_specs=pl.BlockSpec((_MB, 16), lambda i: (i, 0)),
        out_shape=jax.ShapeDtypeStruct((n, 16), jnp.float32),
    )(acc1, hs1, dinv, b1[None, :], W2)

    acc2 = _make_prop(16, 128)(src2, dst2, hs2, zeros16)

    out = pl.pallas_call(
        _tc_c_body,
        grid=grid,
        in_specs=[
            pl.BlockSpec((2, _MB, 16), lambda i: (0, i, 0)),
            pl.BlockSpec((_MB, 16), lambda i: (i, 0)),
            pl.BlockSpec((_MB, 1), lambda i: (i, 0)),
            pl.BlockSpec((1, 16), lambda i: (0, 0)),
        ],
        out_specs=pl.BlockSpec((_MB, 16), lambda i: (i, 0)),
        out_shape=jax.ShapeDtypeStruct((n, 16), jnp.float32),
    )(acc2, hs2, dinv, b2[None, :])

    return out


# speedup vs baseline: 1.4334x; 1.4334x over previous
import functools

import jax
import jax.numpy as jnp
from jax import lax
from jax.experimental import pallas as pl
from jax.experimental.pallas import tpu as pltpu
from jax.experimental.pallas import tpu_sc as plsc

N_NODES = 10000
N_PAD = 10240
E_EDGES = 320000
CHUNKS_PER_TILE = 160
E_PAD = 16 * CHUNKS_PER_TILE * 128
ROWS_PER_TILE = N_PAD // 16
NBUF = 4


@functools.cache
def _mesh():
    return plsc.VectorSubcoreMesh(core_axis_name="c", subcore_axis_name="s")


@functools.cache
def _make_prop(width, k0):
    nb = 2 if width == 128 else 4
    stage_src = width != 128
    k1 = CHUNKS_PER_TILE - k0
    ncores = 1 if k1 == 0 else 2

    scratch = [
        pltpu.VMEM((max(k0, k1), 128) if stage_src else (4, 128), jnp.int32),
        pltpu.VMEM((max(k0, k1), 128) if stage_src else (4, 128), jnp.int32),
        pltpu.VMEM((nb, 128, width), jnp.float32),
        pltpu.VMEM_SHARED((N_PAD, width), jnp.float32),
    ] + [pltpu.SemaphoreType.DMA] * (2 * nb + (0 if stage_src else 8))

    @functools.partial(
        pl.kernel,
        out_type=jax.ShapeDtypeStruct((ncores, N_PAD, width), jnp.float32),
        mesh=_mesh(),
        compiler_params=pltpu.CompilerParams(use_tc_tiling_on_sc=False),
        scratch_types=scratch,
    )
    def prop(src_hbm, dst_hbm, h_hbm, z_hbm, out_hbm, sidx, didx, rows,
             acc_sh, *sems):
        gsem = sems[:nb]
        ssem = sems[nb:2 * nb]
        isem = sems[2 * nb:2 * nb + 4]
        dsem = sems[2 * nb + 4:]
        c = lax.axis_index("c")
        s = lax.axis_index("s")
        my_rows = s * ROWS_PER_TILE
        if ncores == 1:
            cbase = s * k0
            nch = k0
        else:
            cbase = jnp.where(c == 0, s * k0, 16 * k0 + s * k1)
            nch = jnp.where(c == 0, k0, k1)

        def stage():
            if stage_src:
                pltpu.sync_copy(dst_hbm.at[pl.ds(cbase, k1)],
                                didx.at[pl.ds(0, k1)])
                pltpu.sync_copy(src_hbm.at[pl.ds(cbase, k1)],
                                sidx.at[pl.ds(0, k1)])

                @pl.when(c == 0)
                def _():
                    pltpu.sync_copy(dst_hbm.at[pl.ds(cbase + k1, k0 - k1)],
                                    didx.at[pl.ds(k1, k0 - k1)])
                    pltpu.sync_copy(src_hbm.at[pl.ds(cbase + k1, k0 - k1)],
                                    sidx.at[pl.ds(k1, k0 - k1)])
            else:
                pltpu.sync_copy(src_hbm.at[pl.ds(cbase, 2)],
                                sidx.at[pl.ds(0, 2)])
                pltpu.sync_copy(dst_hbm.at[pl.ds(cbase, 2)],
                                didx.at[pl.ds(0, 2)])
                for u in (2, 3):
                    pltpu.async_copy(src_hbm.at[cbase + u], sidx.at[u],
                                     isem[u])
                    pltpu.async_copy(dst_hbm.at[cbase + u], didx.at[u],
                                     dsem[u])

            pltpu.sync_copy(z_hbm, rows.at[0])
            for z in range(ROWS_PER_TILE // 128):
                pltpu.sync_copy(rows.at[0],
                                acc_sh.at[pl.ds(my_rows + z * 128, 128)])
            plsc.subcore_barrier()

        def wait_rows(sem, b):
            pltpu.make_async_copy(h_hbm.at[pl.ds(0, 128)], rows.at[b],
                                  sem).wait()

        def wait_idx(u):
            pltpu.make_async_copy(src_hbm.at[cbase], sidx.at[u],
                                  isem[u]).wait()

        def wait_didx(u):
            pltpu.make_async_copy(dst_hbm.at[cbase], didx.at[u],
                                  dsem[u]).wait()

        def gather(jj, u, b):
            src_idx = sidx.at[jj] if stage_src else sidx.at[u]
            pltpu.async_copy(h_hbm.at[src_idx], rows.at[b], gsem[b])

        unroll = nb if stage_src else 4

        def body(i, _):
            for u in range(unroll):
                j = i * unroll + u
                b = u % nb
                wait_rows(gsem[b], b)
                if not stage_src:
                    @pl.when(j + 4 < nch)
                    def _():
                        pltpu.async_copy(src_hbm.at[cbase + j + 4],
                                         sidx.at[u], isem[u])
                    if u >= 2:
                        wait_didx(u)
                    else:
                        @pl.when(j >= 4)
                        def _():
                            wait_didx(u)
                dchunk = didx.at[j] if stage_src else didx.at[u]
                pltpu.async_copy(rows.at[b], acc_sh.at[dchunk],
                                 ssem[b], add=True)

                @pl.when(j + nb < nch)
                def _():
                    wait_rows(ssem[b], b)
                    if not stage_src:
                        @pl.when(j + 4 < nch)
                        def _():
                            pltpu.async_copy(dst_hbm.at[cbase + j + 4],
                                             didx.at[u], dsem[u])
                        wait_idx((u + nb) % 4)
                    gather(j + nb, (u + nb) % 4, b)

            return 0

        def run():
            stage()
            for b in range(nb):
                gather(b, b, b)
            lax.fori_loop(0, nch // unroll, body, 0)
            for b in range(nb):
                wait_rows(ssem[b], b)
            plsc.subcore_barrier()

            for z in range(ROWS_PER_TILE // 128):
                r0 = my_rows + z * 128
                pltpu.sync_copy(acc_sh.at[pl.ds(r0, 128)],
                                out_hbm.at[c, pl.ds(r0, 128)])

        if ncores == 1:
            pl.when(c == 0)(run)
        else:
            run()

    return prop


@functools.cache
def _make_deg(k0):
    k1 = CHUNKS_PER_TILE - k0

    @functools.partial(
        pl.kernel,
        out_type=jax.ShapeDtypeStruct((2, N_PAD, 16), jnp.float32),
        mesh=_mesh(),
        compiler_params=pltpu.CompilerParams(use_tc_tiling_on_sc=False),
        scratch_types=[
            pltpu.VMEM((max(k0, k1), 128), jnp.int32),
            pltpu.VMEM((128, 16), jnp.float32),
            pltpu.VMEM_SHARED((N_PAD, 16), jnp.float32),
        ] + [pltpu.SemaphoreType.DMA] * NBUF,
    )
    def deg_kernel(dst_hbm, const_hbm, out_hbm, didx, rows, acc_sh, *ssem):
        c = lax.axis_index("c")
        s = lax.axis_index("s")
        my_rows = s * ROWS_PER_TILE
        cbase = jnp.where(c == 0, s * k0, 16 * k0 + s * k1)
        nch = jnp.where(c == 0, k0, k1)

        pltpu.sync_copy(dst_hbm.at[pl.ds(cbase, k1)], didx.at[pl.ds(0, k1)])

        @pl.when(c == 0)
        def _():
            pltpu.sync_copy(dst_hbm.at[pl.ds(cbase + k1, k0 - k1)],
                            didx.at[pl.ds(k1, k0 - k1)])

        pltpu.sync_copy(const_hbm.at[0], rows)
        for z in range(ROWS_PER_TILE // 128):
            pltpu.sync_copy(rows, acc_sh.at[pl.ds(my_rows + z * 128, 128)])
        plsc.subcore_barrier()

        pltpu.sync_copy(const_hbm.at[1], rows)

        def wait(sem):
            pltpu.make_async_copy(out_hbm.at[c, pl.ds(0, 128)], rows,
                                  sem).wait()

        def body(i, _):
            for b in range(NBUF):
                j = i * NBUF + b

                @pl.when(j >= NBUF)
                def _():
                    wait(ssem[b])

                pltpu.async_copy(rows, acc_sh.at[didx.at[j]], ssem[b],
                                 add=True)
            return 0

        lax.fori_loop(0, nch // NBUF, body, 0)
        for b in range(NBUF):
            wait(ssem[b])
        plsc.subcore_barrier()

        for z in range(ROWS_PER_TILE // 128):
            r0 = my_rows + z * 128
            pltpu.sync_copy(acc_sh.at[pl.ds(r0, 128)],
                            out_hbm.at[c, pl.ds(r0, 128)])

    return deg_kernel


def _tc_a_body(x_ref, w1_ref, degp_ref, hs1_ref, dinv_ref):
    t = degp_ref[0] + degp_ref[1]
    deg = t[:, 0:1] + 1.0
    dinv = lax.rsqrt(deg)
    h = jnp.dot(x_ref[...], w1_ref[...], preferred_element_type=jnp.float32)
    hs1_ref[...] = h * dinv
    dinv_ref[...] = dinv


def _tc_b_body(accp_ref, hs1_ref, dinv_ref, b1_ref, w2_ref, hs2_ref):
    t = accp_ref[0] + accp_ref[1] + hs1_ref[...]
    out1 = dinv_ref[...] * t + b1_ref[...]
    h = jnp.maximum(out1, 0.0)
    h2 = jnp.dot(h, w2_ref[...], preferred_element_type=jnp.float32)
    hs2_ref[...] = h2 * dinv_ref[...]


def _tc_c_body(accp_ref, hs2_ref, dinv_ref, b2_ref, out_ref):
    t = accp_ref[0] + accp_ref[1] + hs2_ref[...]
    out2 = dinv_ref[...] * t + b2_ref[...]
    m = jnp.max(out2, axis=1, keepdims=True)
    e = jnp.exp(out2 - m)
    lse = jnp.log(jnp.sum(e, axis=1, keepdims=True))
    out_ref[...] = out2 - m - lse


_MB = 1000


def kernel(x, edge_index, W1, b1, W2, b2):
    n = N_NODES
    pad = E_PAD - E_EDGES
    src = jnp.concatenate([edge_index[0], jnp.zeros((pad,), jnp.int32)])
    dst = jnp.concatenate([edge_index[1], jnp.full((pad,), n, jnp.int32)])
    src2 = src.reshape(E_PAD // 128, 128)
    dst2 = dst.reshape(E_PAD // 128, 128)

    zeros16 = jnp.zeros((128, 16), jnp.float32)
    deg_const = jnp.stack([zeros16, zeros16.at[:, 0].set(1.0)])
    degp = _make_deg(120)(dst2, deg_const)

    grid = (n // _MB,)
    hs1, dinv = pl.pallas_call(
        _tc_a_body,
        grid=grid,
        in_specs=[
            pl.BlockSpec((_MB, 128), lambda i: (i, 0)),
            pl.BlockSpec((128, 128), lambda i: (0, 0)),
            pl.BlockSpec((2, _MB, 16), lambda i: (0, i, 0)),
        ],
        out_specs=[
            pl.BlockSpec((_MB, 128), lambda i: (i, 0)),
            pl.BlockSpec((_MB, 1), lambda i: (i, 0)),
        ],
        out_shape=[
            jax.ShapeDtypeStruct((n, 128), jnp.float32),
            jax.ShapeDtypeStruct((n, 1), jnp.float32),
        ],
    )(x, W1, degp)

    acc1 = _make_prop(128, 156)(src2, dst2, hs1, jnp.zeros((128, 128), jnp.float32))

    hs2 = pl.pallas_call(
        _tc_b_body,
        grid=grid,
        in_specs=[
            pl.BlockSpec((2, _MB, 128), lambda i: (0, i, 0)),
            pl.BlockSpec((_MB, 128), lambda i: (i, 0)),
            pl.BlockSpec((_MB, 1), lambda i: (i, 0)),
            pl.BlockSpec((1, 128), lambda i: (0, 0)),
            pl.BlockSpec((128, 16), lambda i: (0, 0)),
        ],
        out_specs=pl.BlockSpec((_MB, 16), lambda i: (i, 0)),
        out_shape=jax.ShapeDtypeStruct((n, 16), jnp.float32),
    )(acc1, hs1, dinv, b1[None, :], W2)

    acc2 = _make_prop(16, 128)(src2, dst2, hs2, zeros16)

    out = pl.pallas_call(
        _tc_c_body,
        grid=grid,
        in_specs=[
            pl.BlockSpec((2, _MB, 16), lambda i: (0, i, 0)),
            pl.BlockSpec((_MB, 16), lambda i: (i, 0)),
            pl.BlockSpec((_MB, 1), lambda i: (i, 0)),
            pl.BlockSpec((1, 16), lambda i: (0, 0)),
        ],
        out_specs=pl.BlockSpec((_MB, 16), lambda i: (i, 0)),
        out_shape=jax.ShapeDtypeStruct((n, 16), jnp.float32),
    )(acc2, hs2, dinv, b2[None, :])

    return out
